# baseline (speedup 1.0000x reference)
import functools

import jax
import jax.numpy as jnp
from jax.experimental import pallas as pl
from jax.experimental.pallas import tpu as pltpu
from jax.experimental.pallas import tpu_sc as plsc

BATCH = 4096
ENC = 512
NCLUST = 32
K = 25
MIN_CONF = 0.25

BLOCK_ROWS = 512
GRID = BATCH // BLOCK_ROWS


def _block_body(enc_row_ref, enc_full_ref, cat_ref, q_ref, npop_ref,
                n_scr, oh_scr, mg_scr):
    pid = pl.program_id(0)
    E = enc_full_ref[...]

    @pl.when(pid == 0)
    def _prep():
        n_scr[...] = jnp.sum(E * E, axis=1)
        C = cat_ref[...]
        m = jnp.max(C, axis=1, keepdims=True)
        col = jax.lax.broadcasted_iota(jnp.int32, (BATCH, NCLUST), 1)
        first = jnp.min(jnp.where(C == m, col, NCLUST), axis=1, keepdims=True)
        onehot = (col == first).astype(jnp.float32)
        oh_scr[...] = onehot.astype(jnp.bfloat16)
        mg = m[:, 0]
        mg_scr[...] = mg
        sel = (mg >= MIN_CONF).astype(jnp.float32)
        pop = jnp.sum(onehot * sel[:, None], axis=0)
        npop_ref[...] = jnp.sum((pop > 0).astype(jnp.float32)).reshape(1, 1)

    Er = enc_row_ref[...]
    n_full = n_scr[...]
    n_row = n_scr[pl.ds(pid * BLOCK_ROWS, BLOCK_ROWS)]

    G = jax.lax.dot_general(Er, E, (((1,), (1,)), ((), ())),
                            preferred_element_type=jnp.float32)
    D2 = jnp.maximum(n_row[:, None] + n_full[None, :] - 2.0 * G, 0.0)

    bits = jax.lax.bitcast_convert_type(D2, jnp.int32)
    hi16 = (bits >> 16).astype(jnp.int16)

    def tree_count(mask16):
        x = mask16
        w = BATCH
        while w > 128:
            w //= 2
            x = x[:, :w] + x[:, w:2 * w]
        return jnp.sum(x.astype(jnp.int32), axis=1)

    def step_hi(_, carry):
        lo, hi, cnt_a = carry
        mid = lo + ((hi - lo) >> 1)
        mid16 = mid.astype(jnp.int16)
        cnt = tree_count((hi16 <= mid16[:, None]).astype(jnp.int16))
        pred = cnt >= (K + 1)
        return (jnp.where(pred, lo, mid + 1), jnp.where(pred, mid, hi),
                jnp.where(pred, cnt_a, cnt))

    z = jnp.zeros((BLOCK_ROWS,), jnp.int32)
    lo, _, base = jax.lax.fori_loop(
        0, 15, step_hi, (z, jnp.full((BLOCK_ROWS,), 0x7f80, jnp.int32), z))
    rank = (K + 1) - base

    lo16 = lo.astype(jnp.int16)
    key16 = jnp.where(hi16 == lo16[:, None],
                      (bits ^ 0x8000).astype(jnp.int16),
                      jnp.array(32767, jnp.int16))

    def step_lo(_, lohi):
        lo2, hi2 = lohi
        mid = lo2 + ((hi2 - lo2) >> 1)
        mid16 = mid.astype(jnp.int16)
        cnt = tree_count((key16 <= mid16[:, None]).astype(jnp.int16))
        pred = cnt >= rank
        return jnp.where(pred, lo2, mid + 1), jnp.where(pred, mid, hi2)

    s, _ = jax.lax.fori_loop(
        0, 16, step_lo,
        (jnp.full((BLOCK_ROWS,), -32768, jnp.int32),
         jnp.full((BLOCK_ROWS,), 32767, jnp.int32)))

    t_bits = (lo << 16) | ((s + 32768) & 0xFFFF)

    maskb = (bits < t_bits[:, None]).astype(jnp.bfloat16)

    counts = jax.lax.dot_general(maskb, oh_scr[...], (((1,), (0,)), ((), ())),
                                 preferred_element_type=jnp.float32)
    denom = jnp.sum(counts, axis=1, keepdims=True)
    bins = counts / denom
    ent = -jnp.sum(bins * jnp.log(bins + 1e-5), axis=1)

    mg_row = mg_scr[pl.ds(pid * BLOCK_ROWS, BLOCK_ROWS)]
    q_ref[...] = ent * mg_row


@jax.jit
def _per_row_entropy(encodings, categorical):
    q, npop = pl.pallas_call(
        _block_body,
        grid=(GRID,),
        in_specs=[
            pl.BlockSpec((BLOCK_ROWS, ENC), lambda i: (i, 0)),
            pl.BlockSpec((BATCH, ENC), lambda i: (0, 0)),
            pl.BlockSpec((BATCH, NCLUST), lambda i: (0, 0)),
        ],
        out_specs=[
            pl.BlockSpec((BLOCK_ROWS,), lambda i: (i,)),
            pl.BlockSpec((1, 1), lambda i: (0, 0)),
        ],
        out_shape=[
            jax.ShapeDtypeStruct((BATCH,), jnp.float32),
            jax.ShapeDtypeStruct((1, 1), jnp.float32),
        ],
        scratch_shapes=[
            pltpu.VMEM((BATCH,), jnp.float32),
            pltpu.VMEM((BATCH, NCLUST), jnp.bfloat16),
            pltpu.VMEM((BATCH,), jnp.float32),
        ],
    )(encodings, encodings, categorical)
    return q, npop


def _sc_gather(q, idxs):
    info = plsc.get_sparse_core_info()
    nw = info.num_cores * info.num_subcores
    bpw = BATCH // nw
    mesh = plsc.VectorSubcoreMesh(core_axis_name="c", subcore_axis_name="s")

    @functools.partial(
        pl.kernel, mesh=mesh,
        out_type=jax.ShapeDtypeStruct((BATCH,), jnp.float32),
        scratch_types=[
            pltpu.VMEM((bpw,), jnp.int32),
            pltpu.VMEM((bpw,), jnp.float32),
            pltpu.SemaphoreType.DMA,
        ],
    )
    def k(q_hbm, idx_hbm, out_hbm, idx_v, out_v, sem):
        wid = jax.lax.axis_index("s") * info.num_cores + jax.lax.axis_index("c")
        base = wid * bpw
        pltpu.sync_copy(idx_hbm.at[pl.ds(base, bpw)], idx_v)
        pltpu.async_copy(q_hbm.at[idx_v], out_v, sem).wait()
        pltpu.sync_copy(out_v, out_hbm.at[pl.ds(base, bpw)])

    return k(q, idxs)


def kernel(encodings, categorical, random_idxs):
    q, npop = _per_row_entropy(encodings, categorical)
    neighbourhood_entropy = _sc_gather(q, random_idxs)
    return encodings, neighbourhood_entropy, jnp.reshape(npop, ())


# speedup vs baseline: 1.0124x; 1.0124x over previous
import functools

import jax
import jax.numpy as jnp
from jax.experimental import pallas as pl
from jax.experimental.pallas import tpu as pltpu
from jax.experimental.pallas import tpu_sc as plsc

BATCH = 4096
ENC = 512
NCLUST = 32
K = 25
MIN_CONF = 0.25

BLOCK_ROWS = 512
GRID = BATCH // BLOCK_ROWS


def _block_body(enc_row_ref, enc_full_ref, cat_ref, q_ref, npop_ref,
                n_scr, oh_scr, mg_scr):
    pid = pl.program_id(0)
    E = enc_full_ref[...]

    @pl.when(pid == 0)
    def _prep():
        n_scr[...] = jnp.sum(E * E, axis=1)
        C = cat_ref[...]
        m = jnp.max(C, axis=1, keepdims=True)
        col = jax.lax.broadcasted_iota(jnp.int32, (BATCH, NCLUST), 1)
        first = jnp.min(jnp.where(C == m, col, NCLUST), axis=1, keepdims=True)
        onehot = (col == first).astype(jnp.float32)
        oh_scr[...] = onehot.astype(jnp.bfloat16)
        mg = m[:, 0]
        mg_scr[...] = mg
        sel = (mg >= MIN_CONF).astype(jnp.float32)
        pop = jnp.sum(onehot * sel[:, None], axis=0)
        npop_ref[...] = jnp.sum((pop > 0).astype(jnp.float32)).reshape(1, 1)

    Er = enc_row_ref[...]
    n_full = n_scr[...]
    n_row = n_scr[pl.ds(pid * BLOCK_ROWS, BLOCK_ROWS)]

    G = jax.lax.dot_general(Er, E, (((1,), (1,)), ((), ())),
                            preferred_element_type=jnp.float32)
    D2 = jnp.maximum(n_row[:, None] + n_full[None, :] - 2.0 * G, 0.0)

    bits = jax.lax.bitcast_convert_type(D2, jnp.int32)
    hi16 = (bits >> 16).astype(jnp.int16)

    def tree_count(mask16):
        x = mask16
        w = BATCH
        while w > 128:
            w //= 2
            x = x[:, :w] + x[:, w:2 * w]
        return jnp.sum(x.astype(jnp.int32), axis=1)

    def step_hi(_, carry):
        lo, hi, cnt_a = carry
        mid = lo + ((hi - lo) >> 1)
        mid16 = mid.astype(jnp.int16)
        cnt = tree_count((hi16 <= mid16[:, None]).astype(jnp.int16))
        pred = cnt >= (K + 1)
        return (jnp.where(pred, lo, mid + 1), jnp.where(pred, mid, hi),
                jnp.where(pred, cnt_a, cnt))

    z = jnp.zeros((BLOCK_ROWS,), jnp.int32)
    lo, _, base = jax.lax.fori_loop(
        0, 15, step_hi, (z, jnp.full((BLOCK_ROWS,), 0x7f80, jnp.int32), z))
    rank = (K + 1) - base

    lo16 = lo.astype(jnp.int16)
    key16 = jnp.where(hi16 == lo16[:, None],
                      (bits ^ 0x8000).astype(jnp.int16),
                      jnp.array(32767, jnp.int16))

    def step_lo(_, lohi):
        lo2, hi2 = lohi
        mid = lo2 + ((hi2 - lo2) >> 1)
        mid16 = mid.astype(jnp.int16)
        cnt = tree_count((key16 <= mid16[:, None]).astype(jnp.int16))
        pred = cnt >= rank
        return jnp.where(pred, lo2, mid + 1), jnp.where(pred, mid, hi2)

    s, _ = jax.lax.fori_loop(
        0, 16, step_lo,
        (jnp.full((BLOCK_ROWS,), -32768, jnp.int32),
         jnp.full((BLOCK_ROWS,), 32767, jnp.int32)))

    s16 = s.astype(jnp.int16)[:, None]
    in_bucket = hi16 == lo16[:, None]
    maskb = ((hi16 < lo16[:, None]) | (in_bucket & (key16 < s16))
             ).astype(jnp.bfloat16)

    counts = jax.lax.dot_general(maskb, oh_scr[...], (((1,), (0,)), ((), ())),
                                 preferred_element_type=jnp.float32)
    denom = jnp.sum(counts, axis=1, keepdims=True)
    bins = counts / denom
    ent = -jnp.sum(bins * jnp.log(bins + 1e-5), axis=1)

    mg_row = mg_scr[pl.ds(pid * BLOCK_ROWS, BLOCK_ROWS)]
    q_ref[...] = ent * mg_row


@jax.jit
def _per_row_entropy(encodings, categorical):
    q, npop = pl.pallas_call(
        _block_body,
        grid=(GRID,),
        in_specs=[
            pl.BlockSpec((BLOCK_ROWS, ENC), lambda i: (i, 0)),
            pl.BlockSpec((BATCH, ENC), lambda i: (0, 0)),
            pl.BlockSpec((BATCH, NCLUST), lambda i: (0, 0)),
        ],
        out_specs=[
            pl.BlockSpec((BLOCK_ROWS,), lambda i: (i,)),
            pl.BlockSpec((1, 1), lambda i: (0, 0)),
        ],
        out_shape=[
            jax.ShapeDtypeStruct((BATCH,), jnp.float32),
            jax.ShapeDtypeStruct((1, 1), jnp.float32),
        ],
        scratch_shapes=[
            pltpu.VMEM((BATCH,), jnp.float32),
            pltpu.VMEM((BATCH, NCLUST), jnp.bfloat16),
            pltpu.VMEM((BATCH,), jnp.float32),
        ],
    )(encodings, encodings, categorical)
    return q, npop


def _sc_gather(q, idxs):
    info = plsc.get_sparse_core_info()
    nw = info.num_cores * info.num_subcores
    bpw = BATCH // nw
    mesh = plsc.VectorSubcoreMesh(core_axis_name="c", subcore_axis_name="s")

    @functools.partial(
        pl.kernel, mesh=mesh,
        out_type=jax.ShapeDtypeStruct((BATCH,), jnp.float32),
        scratch_types=[
            pltpu.VMEM((bpw,), jnp.int32),
            pltpu.VMEM((bpw,), jnp.float32),
            pltpu.SemaphoreType.DMA,
        ],
    )
    def k(q_hbm, idx_hbm, out_hbm, idx_v, out_v, sem):
        wid = jax.lax.axis_index("s") * info.num_cores + jax.lax.axis_index("c")
        base = wid * bpw
        pltpu.sync_copy(idx_hbm.at[pl.ds(base, bpw)], idx_v)
        pltpu.async_copy(q_hbm.at[idx_v], out_v, sem).wait()
        pltpu.sync_copy(out_v, out_hbm.at[pl.ds(base, bpw)])

    return k(q, idxs)


def kernel(encodings, categorical, random_idxs):
    q, npop = _per_row_entropy(encodings, categorical)
    neighbourhood_entropy = _sc_gather(q, random_idxs)
    return encodings, neighbourhood_entropy, jnp.reshape(npop, ())
